# baseline (speedup 1.0000x reference)
import functools

import jax
import jax.numpy as jnp
import numpy as np
from jax import lax
from jax.experimental import pallas as pl
from jax.experimental.pallas import tpu as pltpu
from jax.experimental.pallas import tpu_sc as plsc

VOCAB = 64
N_LANGS = 16
DIM = 128
KDIM = 2 * DIM
LANES = 16
NW = 8
RPW = DIM // NW
NGROUP = RPW // LANES
NCHUNK = DIM // LANES

_DNUMS = lax.GatherDimensionNumbers(
    offset_dims=(), collapsed_slice_dims=(0,), start_index_map=(0,))


def _lane_bcast(v, ki):
    idx = jnp.full((LANES, 1), ki, jnp.int32)
    return lax.gather(v, idx, _DNUMS, (1,),
                      mode=lax.GatherScatterMode.PROMISE_IN_BOUNDS)


def _embed_fc_body(aux_hbm, char_hbm, lang_hbm, w_hbm, b_hbm,
                   out_hbm, aux_v, char_v, lang_v, w_v, b_v, out_v, sem):
    wid = lax.axis_index("s")

    @pl.when(wid < NW)
    def _work():
        base = wid * RPW

        cp_a = pltpu.async_copy(aux_hbm, aux_v, sem)
        cp_c = pltpu.async_copy(char_hbm, char_v, sem)
        cp_l = pltpu.async_copy(lang_hbm, lang_v, sem)
        cp_w = pltpu.async_copy(w_hbm.at[pl.ds(base * KDIM, RPW * KDIM)],
                                w_v, sem)
        cp_b = pltpu.async_copy(b_hbm.at[pl.ds(base, RPW)], b_v, sem)
        cp_a.wait()
        cp_c.wait()
        cp_l.wait()
        cp_w.wait()
        cp_b.wait()

        cvec = aux_v[pl.ds(0, LANES)]
        lvec = aux_v[pl.ds(LANES, LANES)]
        lane_off = aux_v[pl.ds(2 * LANES, LANES)]
        lane = lax.shift_right_logical(lane_off, 8)

        xoff_c = (cvec << 7) + lane
        xoff_l = (lvec << 7) + lane

        NACC = 4

        for g in range(NGROUP):
            goff = g * LANES * KDIM

            def phase(src, xoff, koff, accs):
                accs = list(accs)
                for c2 in range(NCHUNK):
                    xv = plsc.load_gather(src, [xoff + c2 * LANES])
                    kbase = lane_off + (goff + koff + c2 * LANES)
                    for ki in range(LANES):
                        wcol = plsc.load_gather(w_v, [kbase + ki])
                        a = ki % NACC
                        accs[a] = accs[a] + _lane_bcast(xv, ki) * wcol
                return tuple(accs)

            zero = jnp.zeros((LANES,), jnp.float32)
            accs = (b_v[pl.ds(g * LANES, LANES)],) + (zero,) * (NACC - 1)
            accs = phase(char_v, xoff_c, 0, accs)
            accs = phase(lang_v, xoff_l, DIM, accs)
            out_v[pl.ds(g * LANES, LANES)] = ((accs[0] + accs[1])
                                              + (accs[2] + accs[3]))

        pltpu.sync_copy(out_v, out_hbm.at[pl.ds(base, RPW)])


_embed_fc = functools.partial(
    pl.kernel,
    out_type=jax.ShapeDtypeStruct((DIM,), jnp.float32),
    mesh=plsc.VectorSubcoreMesh(core_axis_name="c", subcore_axis_name="s",
                                num_cores=1),
    compiler_params=pltpu.CompilerParams(needs_layout_passes=False),
    scratch_types=[
        pltpu.VMEM((3 * LANES,), jnp.int32),
        pltpu.VMEM((VOCAB * DIM,), jnp.float32),
        pltpu.VMEM((N_LANGS * DIM,), jnp.float32),
        pltpu.VMEM((RPW * KDIM,), jnp.float32),
        pltpu.VMEM((RPW,), jnp.float32),
        pltpu.VMEM((RPW,), jnp.float32),
        pltpu.SemaphoreType.DMA,
    ],
)(_embed_fc_body)

_LANE_OFF = np.arange(LANES, dtype=np.int32) * KDIM


def kernel(char_idx, lang, char_table, lang_table, W, b):
    ci = jnp.asarray(char_idx, jnp.int32)
    li = jnp.asarray(lang, jnp.int32)
    aux = jnp.concatenate([jnp.full((LANES,), ci, jnp.int32),
                           jnp.full((LANES,), li, jnp.int32),
                           jnp.asarray(_LANE_OFF)])
    out = _embed_fc(aux, char_table.reshape(-1), lang_table.reshape(-1),
                    W.reshape(-1), b)
    return out.reshape(1, DIM)


# speedup vs baseline: 1.0026x; 1.0026x over previous
import functools

import jax
import jax.numpy as jnp
import numpy as np
from jax import lax
from jax.experimental import pallas as pl
from jax.experimental.pallas import tpu as pltpu
from jax.experimental.pallas import tpu_sc as plsc

VOCAB = 64
N_LANGS = 16
DIM = 128
KDIM = 2 * DIM
LANES = 16
NW = 8
RPW = DIM // NW
NCHUNK = DIM // LANES
NACC = 4

_DNUMS = lax.GatherDimensionNumbers(
    offset_dims=(), collapsed_slice_dims=(0,), start_index_map=(0,))


def _lane_bcast(v, ki):
    idx = jnp.full((LANES, 1), ki, jnp.int32)
    return lax.gather(v, idx, _DNUMS, (1,),
                      mode=lax.GatherScatterMode.PROMISE_IN_BOUNDS)


def _embed_fc_body(aux_hbm, char_hbm, lang_hbm, wt_hbm, b_hbm,
                   out_hbm, aux_v, char_v, lang_v, w_v, b_v, out_v, sem):
    wid = lax.axis_index("s")

    @pl.when(wid < NW)
    def _work():
        base = wid * RPW

        cp_a = pltpu.async_copy(aux_hbm, aux_v, sem)
        cp_c = pltpu.async_copy(char_hbm, char_v, sem)
        cp_l = pltpu.async_copy(lang_hbm, lang_v, sem)
        cp_w = pltpu.async_copy(wt_hbm.at[wid], w_v, sem)
        cp_b = pltpu.async_copy(b_hbm.at[pl.ds(base, RPW)], b_v, sem)
        cp_a.wait()
        cp_c.wait()
        cp_l.wait()
        cp_w.wait()
        cp_b.wait()

        cvec = aux_v[pl.ds(0, LANES)]
        lvec = aux_v[pl.ds(LANES, LANES)]
        lane_off = aux_v[pl.ds(2 * LANES, LANES)]
        lane = lax.shift_right_logical(lane_off, 8)

        xoff_c = (cvec << 7) + lane
        xoff_l = (lvec << 7) + lane

        def phase(src, xoff, koff, accs):
            accs = list(accs)
            for c2 in range(NCHUNK):
                xv = plsc.load_gather(src, [xoff + c2 * LANES])
                for ki in range(LANES):
                    k = koff + c2 * LANES + ki
                    wcol = w_v[k, :]
                    a = ki % NACC
                    accs[a] = accs[a] + _lane_bcast(xv, ki) * wcol
            return tuple(accs)

        zero = jnp.zeros((LANES,), jnp.float32)
        accs = (b_v[...],) + (zero,) * (NACC - 1)
        accs = phase(char_v, xoff_c, 0, accs)
        accs = phase(lang_v, xoff_l, DIM, accs)
        out_v[...] = (accs[0] + accs[1]) + (accs[2] + accs[3])

        pltpu.sync_copy(out_v, out_hbm.at[pl.ds(base, RPW)])


_embed_fc = functools.partial(
    pl.kernel,
    out_type=jax.ShapeDtypeStruct((DIM,), jnp.float32),
    mesh=plsc.VectorSubcoreMesh(core_axis_name="c", subcore_axis_name="s",
                                num_cores=1),
    compiler_params=pltpu.CompilerParams(needs_layout_passes=False),
    scratch_types=[
        pltpu.VMEM((3 * LANES,), jnp.int32),
        pltpu.VMEM((VOCAB * DIM,), jnp.float32),
        pltpu.VMEM((N_LANGS * DIM,), jnp.float32),
        pltpu.VMEM((KDIM, LANES), jnp.float32),
        pltpu.VMEM((RPW,), jnp.float32),
        pltpu.VMEM((RPW,), jnp.float32),
        pltpu.SemaphoreType.DMA,
    ],
)(_embed_fc_body)

_LANE_OFF = np.arange(LANES, dtype=np.int32) * KDIM


def kernel(char_idx, lang, char_table, lang_table, W, b):
    ci = jnp.asarray(char_idx, jnp.int32)
    li = jnp.asarray(lang, jnp.int32)
    aux = jnp.concatenate([jnp.full((LANES,), ci, jnp.int32),
                           jnp.full((LANES,), li, jnp.int32),
                           jnp.asarray(_LANE_OFF)])
    wt = W.reshape(NW, RPW, KDIM).transpose(0, 2, 1)
    out = _embed_fc(aux, char_table.reshape(-1), lang_table.reshape(-1),
                    wt, b)
    return out.reshape(1, DIM)


# speedup vs baseline: 1.0800x; 1.0773x over previous
import functools

import jax
import jax.numpy as jnp
import numpy as np
from jax import lax
from jax.experimental import pallas as pl
from jax.experimental.pallas import tpu as pltpu
from jax.experimental.pallas import tpu_sc as plsc

VOCAB = 64
N_LANGS = 16
DIM = 128
KDIM = 2 * DIM
LANES = 16
NW = 8
RPW = DIM // NW
NCHUNK = DIM // LANES

_DNUMS = lax.GatherDimensionNumbers(
    offset_dims=(), collapsed_slice_dims=(0,), start_index_map=(0,))


def _perm(v, idxvec):
    return lax.gather(v, idxvec.reshape(LANES, 1), _DNUMS, (1,),
                      mode=lax.GatherScatterMode.PROMISE_IN_BOUNDS)


def _embed_fc_body(aux_hbm, char_hbm, lang_hbm, w_hbm, b_hbm,
                   out_hbm, aux_v, char_v, lang_v, w_v, b_v, out_v, sem):
    wid = lax.axis_index("s")

    @pl.when(wid < NW)
    def _work():
        base = wid * RPW

        cp_a = pltpu.async_copy(aux_hbm, aux_v, sem)
        cp_c = pltpu.async_copy(char_hbm, char_v, sem)
        cp_l = pltpu.async_copy(lang_hbm, lang_v, sem)
        cp_w = pltpu.async_copy(w_hbm.at[pl.ds(base, RPW), :], w_v, sem)
        cp_b = pltpu.async_copy(b_hbm.at[pl.ds(base, RPW)], b_v, sem)
        cp_a.wait()
        cp_c.wait()
        cp_l.wait()
        cp_w.wait()
        cp_b.wait()

        cvec = aux_v[pl.ds(0, LANES)]
        lvec = aux_v[pl.ds(LANES, LANES)]
        lane_off = aux_v[pl.ds(2 * LANES, LANES)]
        lane = lax.shift_right_logical(lane_off, 8)

        xoff_c = (cvec << 7) + lane
        xoff_l = (lvec << 7) + lane

        accs = [jnp.zeros((LANES,), jnp.float32) for _ in range(RPW)]
        for phase, (src, xoff) in enumerate(((char_v, xoff_c),
                                             (lang_v, xoff_l))):
            for c2 in range(NCHUNK):
                xv = plsc.load_gather(src, [xoff + c2 * LANES])
                col = phase * DIM + c2 * LANES
                for r in range(RPW):
                    accs[r] = accs[r] + xv * w_v[r, pl.ds(col, LANES)]

        vecs = accs
        for m in (1, 2, 4, 8):
            pidx = lane ^ m
            sel = (lane & m) == 0
            nxt = []
            for j in range(0, len(vecs), 2):
                a = vecs[j] + _perm(vecs[j], pidx)
                bb = vecs[j + 1] + _perm(vecs[j + 1], pidx)
                nxt.append(jnp.where(sel, a, bb))
            vecs = nxt
        out_v[...] = vecs[0] + b_v[...]

        pltpu.sync_copy(out_v, out_hbm.at[pl.ds(base, RPW)])


_embed_fc = functools.partial(
    pl.kernel,
    out_type=jax.ShapeDtypeStruct((DIM,), jnp.float32),
    mesh=plsc.VectorSubcoreMesh(core_axis_name="c", subcore_axis_name="s",
                                num_cores=1),
    compiler_params=pltpu.CompilerParams(needs_layout_passes=False),
    scratch_types=[
        pltpu.VMEM((3 * LANES,), jnp.int32),
        pltpu.VMEM((VOCAB * DIM,), jnp.float32),
        pltpu.VMEM((N_LANGS * DIM,), jnp.float32),
        pltpu.VMEM((RPW, KDIM), jnp.float32),
        pltpu.VMEM((RPW,), jnp.float32),
        pltpu.VMEM((RPW,), jnp.float32),
        pltpu.SemaphoreType.DMA,
    ],
)(_embed_fc_body)

_LANE_OFF = np.arange(LANES, dtype=np.int32) * KDIM


def kernel(char_idx, lang, char_table, lang_table, W, b):
    ci = jnp.asarray(char_idx, jnp.int32)
    li = jnp.asarray(lang, jnp.int32)
    aux = jnp.concatenate([jnp.full((LANES,), ci, jnp.int32),
                           jnp.full((LANES,), li, jnp.int32),
                           jnp.asarray(_LANE_OFF)])
    out = _embed_fc(aux, char_table.reshape(-1), lang_table.reshape(-1),
                    W, b)
    return out.reshape(1, DIM)


# speedup vs baseline: 1.0898x; 1.0091x over previous
import functools

import jax
import jax.numpy as jnp
import numpy as np
from jax import lax
from jax.experimental import pallas as pl
from jax.experimental.pallas import tpu as pltpu
from jax.experimental.pallas import tpu_sc as plsc

VOCAB = 64
N_LANGS = 16
DIM = 128
KDIM = 2 * DIM
LANES = 16
NW = 8
RPW = DIM // NW
NCHUNK = DIM // LANES

_DNUMS = lax.GatherDimensionNumbers(
    offset_dims=(), collapsed_slice_dims=(0,), start_index_map=(0,))


def _perm(v, idxvec):
    return lax.gather(v, idxvec.reshape(LANES, 1), _DNUMS, (1,),
                      mode=lax.GatherScatterMode.PROMISE_IN_BOUNDS)


def _embed_fc_body(aux_hbm, char_hbm, lang_hbm, w_hbm, b_hbm,
                   out_hbm, aux_v, char_v, lang_v, w_v, b_v, out_v, sem):
    wid = lax.axis_index("s")

    @pl.when(wid < NW)
    def _work():
        base = wid * RPW

        cp_a = pltpu.async_copy(aux_hbm, aux_v, sem)
        cp_c = pltpu.async_copy(char_hbm, char_v, sem)
        cp_l = pltpu.async_copy(lang_hbm, lang_v, sem)
        cp_w = pltpu.async_copy(w_hbm.at[pl.ds(base, RPW), :], w_v, sem)
        cp_b = pltpu.async_copy(b_hbm.at[pl.ds(base, RPW)], b_v, sem)
        cp_a.wait()
        cp_c.wait()
        cp_l.wait()
        cp_w.wait()
        cp_b.wait()

        cvec = aux_v[pl.ds(0, LANES)]
        lvec = aux_v[pl.ds(LANES, LANES)]
        lane_off = aux_v[pl.ds(2 * LANES, LANES)]
        lane = lax.shift_right_logical(lane_off, 8)

        xoff_c = (cvec << 7) + lane
        xoff_l = (lvec << 7) + lane

        accs = [jnp.zeros((LANES,), jnp.float32) for _ in range(RPW)]
        for phase, (src, xoff) in enumerate(((char_v, xoff_c),
                                             (lang_v, xoff_l))):
            for c2 in range(NCHUNK):
                xv = plsc.load_gather(src, [xoff + c2 * LANES])
                col = phase * DIM + c2 * LANES
                for r in range(RPW):
                    accs[r] = accs[r] + xv * w_v[r, pl.ds(col, LANES)]

        vecs = accs
        for m in (1, 2, 4, 8):
            pidx = lane ^ m
            sel = (lane & m) == 0
            nxt = []
            for j in range(0, len(vecs), 2):
                a = vecs[j] + _perm(vecs[j], pidx)
                bb = vecs[j + 1] + _perm(vecs[j + 1], pidx)
                nxt.append(jnp.where(sel, a, bb))
            vecs = nxt
        out_v[...] = vecs[0] + b_v[...]

        pltpu.sync_copy(out_v, out_hbm.at[pl.ds(base, RPW)])


_embed_fc = functools.partial(
    pl.kernel,
    out_type=jax.ShapeDtypeStruct((DIM,), jnp.float32),
    mesh=plsc.VectorSubcoreMesh(core_axis_name="c", subcore_axis_name="s",
                                num_cores=1, num_subcores=NW),
    compiler_params=pltpu.CompilerParams(needs_layout_passes=False),
    scratch_types=[
        pltpu.VMEM((3 * LANES,), jnp.int32),
        pltpu.VMEM((VOCAB * DIM,), jnp.float32),
        pltpu.VMEM((N_LANGS * DIM,), jnp.float32),
        pltpu.VMEM((RPW, KDIM), jnp.float32),
        pltpu.VMEM((RPW,), jnp.float32),
        pltpu.VMEM((RPW,), jnp.float32),
        pltpu.SemaphoreType.DMA,
    ],
)(_embed_fc_body)

_LANE_OFF = np.arange(LANES, dtype=np.int32) * KDIM


def kernel(char_idx, lang, char_table, lang_table, W, b):
    ci = jnp.asarray(char_idx, jnp.int32)
    li = jnp.asarray(lang, jnp.int32)
    aux = jnp.concatenate([jnp.full((LANES,), ci, jnp.int32),
                           jnp.full((LANES,), li, jnp.int32),
                           jnp.asarray(_LANE_OFF)])
    out = _embed_fc(aux, char_table.reshape(-1), lang_table.reshape(-1),
                    W, b)
    return out.reshape(1, DIM)
